# baseline (speedup 1.0000x reference)
import functools
import math

import jax
import jax.numpy as jnp
from jax import lax
from jax.experimental import pallas as pl
from jax.experimental.pallas import tpu as pltpu
from jax.experimental.pallas import tpu_sc as plsc

N = 256
CUTOFF = 10
LN2 = 0.6931471805599453
L = 16

_DISC = tuple(1.0 / math.log2(r + 2) for r in range(CUTOFF))

_LOG1P = (9.97503255e-06, 9.99235484e-01, -4.90230723e-01, 2.85272681e-01,
          -1.31581825e-01, 3.04490045e-02)

_SC_MESH = plsc.VectorSubcoreMesh(core_axis_name="c", subcore_axis_name="s")


@functools.partial(
    pl.kernel,
    mesh=_SC_MESH,
    out_type=jax.ShapeDtypeStruct((2 * L,), jnp.float32),
    scratch_types=[
        pltpu.VMEM((N + L,), jnp.float32),
        pltpu.VMEM((N + L,), jnp.float32),
        pltpu.VMEM((N + L,), jnp.float32),
        pltpu.VMEM((N + L,), jnp.float32),
        pltpu.VMEM((L,), jnp.float32),
        pltpu.VMEM((L,), jnp.float32),
        pltpu.VMEM_SHARED((N,), jnp.float32),
        pltpu.VMEM_SHARED((N,), jnp.float32),
        pltpu.VMEM_SHARED((N,), jnp.float32),
        pltpu.VMEM_SHARED((N,), jnp.float32),
    ],
)
def _sc_loss(o_hbm, s_hbm, out_hbm, o_buf, s_buf, g_buf, d_buf, tmp_v,
             out_v, d_sh, g_sh, pair_sh, idcg_sh):
    cid = lax.axis_index("c")
    sid = lax.axis_index("s")
    iota = lax.iota(jnp.int32, L)
    local = sid * L
    mine_idx = local + iota

    pltpu.sync_copy(o_hbm, o_buf.at[pl.ds(0, N)])
    pltpu.sync_copy(o_hbm.at[pl.ds(0, L)], o_buf.at[pl.ds(N, L)])
    pltpu.sync_copy(s_hbm, s_buf.at[pl.ds(0, N)])
    pltpu.sync_copy(s_hbm.at[pl.ds(0, L)], s_buf.at[pl.ds(N, L)])

    my_o = o_buf[pl.ds(local, L)]
    my_s = s_buf[pl.ds(local, L)]

    def rank_body(t, carry):
        acc_o, acc_s = carry
        vo = o_buf[pl.ds(t, L)]
        vs = s_buf[pl.ds(t, L)]
        kg = t + iota
        kidx = jnp.where(kg >= N, kg - N, kg)
        tie_lt = kidx < mine_idx
        acc_o = (acc_o + jnp.where(vo > my_o, 1, 0)
                 + jnp.where((vo == my_o) & tie_lt, 1, 0))
        acc_s = (acc_s + jnp.where(vs > my_s, 1, 0)
                 + jnp.where((vs == my_s) & tie_lt, 1, 0))
        return acc_o, acc_s

    zero_i = jnp.zeros((L,), jnp.int32)
    rank_o, rank_s = lax.fori_loop(0, N, rank_body, (zero_i, zero_i))

    def disc(rank):
        d = jnp.zeros((L,), jnp.float32)
        for r in range(CUTOFF):
            d = jnp.where(rank == r, _DISC[r], d)
        return d

    g_chunk = jnp.exp(my_s * LN2) - 1.0
    tmp_v[...] = disc(rank_o)
    pltpu.sync_copy(tmp_v, d_sh.at[pl.ds(local, L)])
    tmp_v[...] = g_chunk
    pltpu.sync_copy(tmp_v, g_sh.at[pl.ds(local, L)])
    tmp_v[...] = g_chunk * disc(rank_s)
    pltpu.sync_copy(tmp_v, idcg_sh.at[pl.ds(local, L)])
    plsc.subcore_barrier()

    pltpu.sync_copy(g_sh, g_buf.at[pl.ds(0, N)])
    pltpu.sync_copy(g_sh.at[pl.ds(0, L)], g_buf.at[pl.ds(N, L)])
    pltpu.sync_copy(d_sh, d_buf.at[pl.ds(0, N)])
    pltpu.sync_copy(d_sh.at[pl.ds(0, L)], d_buf.at[pl.ds(N, L)])
    my_g = g_buf[pl.ds(local, L)]
    my_d = d_buf[pl.ds(local, L)]

    def pair_body(t, acc):
        oj = o_buf[pl.ds(t, L)]
        sj = s_buf[pl.ds(t, L)]
        gj = g_buf[pl.ds(t, L)]
        dj = d_buf[pl.ds(t, L)]
        diff = my_o - oj
        x = jnp.where(my_s > sj, -diff, diff)
        u = jnp.exp(-jnp.abs(x))
        p = jnp.full((L,), _LOG1P[-1], jnp.float32)
        for c in _LOG1P[-2::-1]:
            p = p * u + c
        bce = jnp.minimum(jnp.maximum(x, 0.0) + p, 100.0)
        w = jnp.abs(my_g - gj) * jnp.abs(my_d - dj)
        w = jnp.where(my_o != oj, w, 0.0)
        return acc + bce * w

    base_t = cid * (N // 2)
    part = lax.fori_loop(base_t, base_t + N // 2, pair_body,
                         jnp.zeros((L,), jnp.float32))
    tmp_v[...] = part
    pltpu.sync_copy(tmp_v, pair_sh.at[pl.ds(local, L)])
    plsc.subcore_barrier()

    pltpu.sync_copy(pair_sh, o_buf.at[pl.ds(0, N)])
    pltpu.sync_copy(idcg_sh, s_buf.at[pl.ds(0, N)])
    def red_body(k, carry):
        tot, itot = carry
        return (tot + o_buf[pl.ds(k * L, L)],
                itot + s_buf[pl.ds(k * L, L)])

    zero_f = jnp.zeros((L,), jnp.float32)
    total, itotal = lax.fori_loop(0, N // L, red_body, (zero_f, zero_f))

    g_buf[pl.ds(L, L)] = jnp.zeros((L,), jnp.float32)

    def lane_sum(vec):
        v = vec
        for sh in (8, 4, 2, 1):
            g_buf[pl.ds(0, L)] = v
            v = g_buf[pl.ds(0, L)] + g_buf[pl.ds(sh, L)]
        return v

    tsum = lane_sum(total)
    isum = lane_sum(itotal)
    out_v[...] = tsum / (isum * float(N))

    @pl.when(sid == 0)
    def _():
        pltpu.sync_copy(out_v, out_hbm.at[pl.ds(cid * L, L)])


def kernel(outputs, scores):
    out = _sc_loss(outputs.reshape(-1), scores.reshape(-1))
    return out[0] + out[L]


# speedup vs baseline: 1.2362x; 1.2362x over previous
import functools
import jax
import jax.numpy as jnp
from jax import lax
from jax.experimental import pallas as pl
from jax.experimental.pallas import tpu as pltpu
from jax.experimental.pallas import tpu_sc as plsc

N = 256
CUTOFF = 10
LN2 = 0.6931471805599453
L = 16

_SC_MESH = plsc.VectorSubcoreMesh(core_axis_name="c", subcore_axis_name="s")


@functools.partial(
    pl.kernel,
    mesh=_SC_MESH,
    out_type=jax.ShapeDtypeStruct((N,), jnp.float32),
    scratch_types=[pltpu.VMEM((N,), jnp.float32)],
)
def _sc_stage(o_hbm, out_hbm, buf):
    cid = lax.axis_index("c")
    sid = lax.axis_index("s")

    @pl.when((cid == 0) & (sid == 0))
    def _():
        pltpu.sync_copy(o_hbm, buf)
        pltpu.sync_copy(buf, out_hbm)


def _combine_kernel(o_col, o_row, s_col, s_row, out_ref):
    oc = o_col[...]
    orow = o_row[...]
    sc = s_col[...]
    srow = s_row[...]
    ii = jax.lax.broadcasted_iota(jnp.int32, (N, N), 0)
    kk = jax.lax.broadcasted_iota(jnp.int32, (N, N), 1)
    cmp_i = jnp.where(orow > oc, 1.0, 0.0) + jnp.where((orow == oc) & (kk < ii), 1.0, 0.0)
    rank_col = jnp.sum(cmp_i, axis=1, keepdims=True)
    cmp_j = jnp.where(oc > orow, 1.0, 0.0) + jnp.where((oc == orow) & (ii < kk), 1.0, 0.0)
    rank_row = jnp.sum(cmp_j, axis=0, keepdims=True)
    cmp_s = jnp.where(srow > sc, 1.0, 0.0) + jnp.where((srow == sc) & (kk < ii), 1.0, 0.0)
    rank_s = jnp.sum(cmp_s, axis=1, keepdims=True)

    def disc(rank):
        return jnp.where(rank < CUTOFF, LN2 / jnp.log(rank + 2.0), 0.0)

    d_col = disc(rank_col)
    d_row = disc(rank_row)
    d_s = disc(rank_s)
    g_col = jnp.exp(sc * LN2) - 1.0
    g_row = jnp.exp(srow * LN2) - 1.0
    idcg = jnp.sum(g_col * d_s, axis=(0, 1), keepdims=True)
    diff = oc - orow
    logits = jax.nn.sigmoid(diff)
    log_p = jnp.maximum(jnp.log(logits), -100.0)
    log_1mp = jnp.maximum(jnp.log(1.0 - logits), -100.0)
    labels = jnp.where(sc > srow, 1.0, 0.0)
    bce = -(labels * log_p + (1.0 - labels) * log_1mp)
    w = (jnp.abs(g_col - g_row) * jnp.abs(d_col - d_row)
         * jnp.where(oc != orow, 1.0, 0.0))
    total = jnp.sum(bce * w, axis=(0, 1), keepdims=True)
    out_ref[...] = total / (idcg * N)


def kernel(outputs, scores):
    o = _sc_stage(outputs.reshape(-1))
    s = scores.reshape(-1)
    loss = pl.pallas_call(
        _combine_kernel,
        out_shape=jax.ShapeDtypeStruct((1, 1), jnp.float32),
    )(o.reshape(N, 1), o.reshape(1, N), s.reshape(N, 1), s.reshape(1, N))
    return loss.reshape(())
